# baseline (speedup 1.0000x reference)
import jax
import jax.numpy as jnp
from jax import lax
from jax.experimental import pallas as pl
from jax.experimental.pallas import tpu as pltpu
from jax.experimental.pallas import tpu_sc as plsc

N = 10000
F = 64
E = 320000
K = 10
NSC = 16
NCORE = 2
NW = NSC * NCORE
NPAD = 10240
RPT = NPAD // NW
HROWS = NPAD // NCORE
ZPT = NPAD // NSC
SH = 160
NSH = RPT // SH
C = 128
NCH = 160
EPT_PAD = NCH * C
QF = 16
NQ = F // QF
RING = 8
DIST = RING // 2
NG = NCH // RING


def _rsqrt16(a):
    i = lax.bitcast_convert_type(a, jnp.int32)
    i = jnp.int32(0x5F3759DF) - lax.shift_right_arithmetic(i, 1)
    y = lax.bitcast_convert_type(i, jnp.float32)
    for _ in range(4):
        y = y * (1.5 - 0.5 * a * y * y)
    return y


def _mlp_body(x_ref, w1_ref, b1_ref, w2_ref, b2_ref, o_ref):
    h = jnp.dot(x_ref[...], w1_ref[...], preferred_element_type=jnp.float32)
    h = jnp.maximum(h + b1_ref[...], 0.0)
    o = jnp.dot(h, w2_ref[...], preferred_element_type=jnp.float32)
    o_ref[...] = o + b2_ref[...]


def _mlp(x, W1, b1, W2, b2):
    BR = 400
    return pl.pallas_call(
        _mlp_body,
        grid=(N // BR,),
        in_specs=[
            pl.BlockSpec((BR, 128), lambda i: (i, 0)),
            pl.BlockSpec((128, 256), lambda i: (0, 0)),
            pl.BlockSpec((1, 256), lambda i: (0, 0)),
            pl.BlockSpec((256, F), lambda i: (0, 0)),
            pl.BlockSpec((1, F), lambda i: (0, 0)),
        ],
        out_specs=pl.BlockSpec((BR, F), lambda i: (i, 0)),
        out_shape=jax.ShapeDtypeStruct((NPAD, F), jnp.float32),
    )(x, W1, b1.reshape(1, 256), W2, b2.reshape(1, F))


def _lsm_body(x_ref, o_ref):
    v = x_ref[...]
    m = jnp.max(v, axis=1, keepdims=True)
    e = jnp.exp(v - m)
    s = jnp.sum(e, axis=1, keepdims=True)
    o_ref[...] = v - m - jnp.log(s)


def _log_softmax(x):
    BR = 400
    return pl.pallas_call(
        _lsm_body,
        grid=(N // BR,),
        in_specs=[pl.BlockSpec((BR, F), lambda i: (i, 0))],
        out_specs=pl.BlockSpec((BR, F), lambda i: (i, 0)),
        out_shape=jax.ShapeDtypeStruct((N, F), jnp.float32),
    )(x)


def _sc_body(h_hbm, rowi_hbm, coli_hbm, flg_hbm,
             out_hbm, u0_hbm, u1_hbm, u2_hbm, u3_hbm,
             sx0_hbm, sx1_hbm, sx2_hbm, sx3_hbm,
             S_sh, rowi_v, coli_v, g_v, z_v,
             S0_v, S1_v, S2_v, S3_v, dinv_v, fb_v,
             hbuf_v, xkb_v, ub0_v, ub1_v, ub2_v, ub3_v, gsem, ssem):
    u_hbm = [u0_hbm, u1_hbm, u2_hbm, u3_hbm]
    sx_hbm = [sx0_hbm, sx1_hbm, sx2_hbm, sx3_hbm]
    Sq_v = [S0_v, S1_v, S2_v, S3_v]
    ub_v = [ub0_v, ub1_v, ub2_v, ub3_v]
    cc_ = lax.axis_index("c")
    s_ = lax.axis_index("s")
    w = cc_ * NSC + s_
    rbase = w * RPT
    zb = s_ * ZPT
    pxbase = (1 - cc_) * HROWS + s_ * RPT

    pltpu.sync_copy(rowi_hbm.at[s_], rowi_v)
    pltpu.sync_copy(coli_hbm.at[s_], coli_v)

    zv = jnp.zeros((QF,), jnp.float32)
    ov = jnp.full((QF,), 1.0, jnp.float32)

    def _fill(i, c):
        g_v[0, i, :] = ov
        z_v[i, :] = zv
        return c
    lax.fori_loop(0, C, _fill, 0)

    def _zero_zone():
        for z in range(ZPT // C):
            pltpu.sync_copy(z_v, S_sh.at[pl.ds(zb + z * C, C)])

    def _zero_rows320(base):
        pltpu.sync_copy(z_v, S_sh.at[pl.ds(base, C)])
        pltpu.sync_copy(z_v, S_sh.at[pl.ds(base + C, C)])
        pltpu.sync_copy(z_v.at[pl.ds(0, 64)], S_sh.at[pl.ds(base + 2 * C, 64)])

    def _flag_write(slot, val):
        @pl.when(s_ == 0)
        def _():
            fb_v[0, :] = jnp.full((QF,), val, jnp.int32)
            pltpu.sync_copy(fb_v.at[0], flg_hbm.at[slot])

    def _flag_poll(slot, val):
        @pl.when(s_ == 0)
        def _():
            def cond(carry):
                return carry < val

            def body(carry):
                pltpu.sync_copy(flg_hbm.at[slot], fb_v.at[1])
                return jnp.min(fb_v[1, :])
            lax.while_loop(cond, body, jnp.int32(-2147483647))

    _zero_zone()
    plsc.subcore_barrier()

    def _dgrp(jg, c):
        j0 = jg * RING
        for b in range(RING):
            j = j0 + b

            @pl.when(j >= RING)
            def _():
                pltpu.make_async_copy(
                    g_v.at[0], S_sh.at[rowi_v.at[j - RING]],
                    ssem.at[b]).wait()
            pltpu.async_copy(
                g_v.at[0], S_sh.at[rowi_v.at[j]], ssem.at[b], add=True)
        return c
    lax.fori_loop(0, NG, _dgrp, 0)
    for b in range(RING):
        j = NCH - RING + b
        pltpu.make_async_copy(
            g_v.at[0], S_sh.at[rowi_v.at[j]], ssem.at[b]).wait()
    plsc.subcore_barrier()

    pltpu.sync_copy(S_sh.at[pl.ds(rbase, RPT)], S0_v)
    plsc.subcore_barrier()
    _zero_zone()

    def _dinv(r, c):
        v = S0_v[r, :] + 1.0
        dinv_v[r, :] = _rsqrt16(v)
        return c
    lax.fori_loop(0, RPT, _dinv, 0)
    plsc.subcore_barrier()

    def _uinit(sc, c):
        gb = rbase + sc * SH
        pltpu.sync_copy(h_hbm.at[pl.ds(gb, SH)], hbuf_v)

        def _row(r, ccx):
            dv = dinv_v[sc * SH + r, :]
            for q in range(NQ):
                ub_v[q][r, :] = hbuf_v[r, pl.ds(q * QF, QF)] * dv
            return ccx
        lax.fori_loop(0, SH, _row, 0)
        for q in range(NQ):
            pltpu.sync_copy(ub_v[q], u_hbm[q].at[pl.ds(gb, SH)])
        return c
    lax.fori_loop(0, NSH, _uinit, 0)
    plsc.subcore_barrier()
    _flag_write(2 + cc_, jnp.int32(1))

    def _prime(u_ref):
        for b in range(DIST):
            pltpu.async_copy(u_ref.at[coli_v.at[b]], g_v.at[b], gsem.at[b])

    def _quarter_pass(u_ref):
        def _grp(jg, c):
            j0 = jg * RING
            for b in range(RING):
                j = j0 + b
                pltpu.make_async_copy(
                    u_ref.at[coli_v.at[j]], g_v.at[b], gsem.at[b]).wait()
                pltpu.async_copy(
                    g_v.at[b], S_sh.at[rowi_v.at[j]], ssem.at[b], add=True)
                bp = (b + DIST) % RING
                jold = j - DIST
                jp = j + DIST

                @pl.when(jold >= 0)
                def _():
                    pltpu.make_async_copy(
                        g_v.at[bp], S_sh.at[rowi_v.at[jold]],
                        ssem.at[bp]).wait()

                @pl.when(jp < NCH)
                def _():
                    pltpu.async_copy(
                        u_ref.at[coli_v.at[jp]], g_v.at[bp], gsem.at[bp])
            return c
        lax.fori_loop(0, NG, _grp, 0)
        for i in range(DIST):
            j = NCH - DIST + i
            b = j % RING
            pltpu.make_async_copy(
                g_v.at[b], S_sh.at[rowi_v.at[j]], ssem.at[b]).wait()
        plsc.subcore_barrier()

    def _export(q):
        pltpu.sync_copy(S_sh.at[pl.ds(rbase, RPT)], Sq_v[q])
        pltpu.sync_copy(S_sh.at[pl.ds(pxbase, RPT)],
                        sx_hbm[q].at[pl.ds(pxbase, RPT)])
        _zero_rows320(rbase)
        _zero_rows320(pxbase)

    def _two_passes(qa, qb):
        _prime(u_hbm[qa])
        _quarter_pass(u_hbm[qa])
        _prime(u_hbm[qb])
        _export(qa)
        plsc.subcore_barrier()
        _quarter_pass(u_hbm[qb])
        _export(qb)
        plsc.subcore_barrier()

    def _round(k, c):
        _flag_poll(3 - cc_, k + 1)
        plsc.subcore_barrier()

        @pl.when(cc_ == 0)
        def _():
            _two_passes(0, 1)

        @pl.when(cc_ == 1)
        def _():
            _two_passes(2, 3)

        pltpu.async_copy(h_hbm.at[pl.ds(rbase, SH)], hbuf_v, gsem.at[4])
        for q in range(NQ):
            pltpu.async_copy(u_hbm[q].at[pl.ds(rbase, SH)], ub_v[q],
                             gsem.at[q])

        _flag_write(cc_, k + 1)
        _flag_poll(1 - cc_, k + 1)
        plsc.subcore_barrier()

        @pl.when(cc_ == 0)
        def _():
            pltpu.async_copy(sx_hbm[2].at[pl.ds(rbase, RPT)], Sq_v[2],
                             gsem.at[5])
            pltpu.async_copy(sx_hbm[3].at[pl.ds(rbase, RPT)], Sq_v[3],
                             gsem.at[6])
            pltpu.make_async_copy(sx_hbm[2].at[pl.ds(rbase, RPT)], Sq_v[2],
                                  gsem.at[5]).wait()
            pltpu.make_async_copy(sx_hbm[3].at[pl.ds(rbase, RPT)], Sq_v[3],
                                  gsem.at[6]).wait()

        @pl.when(cc_ == 1)
        def _():
            pltpu.async_copy(sx_hbm[0].at[pl.ds(rbase, RPT)], Sq_v[0],
                             gsem.at[5])
            pltpu.async_copy(sx_hbm[1].at[pl.ds(rbase, RPT)], Sq_v[1],
                             gsem.at[6])
            pltpu.make_async_copy(sx_hbm[0].at[pl.ds(rbase, RPT)], Sq_v[0],
                                  gsem.at[5]).wait()
            pltpu.make_async_copy(sx_hbm[1].at[pl.ds(rbase, RPT)], Sq_v[1],
                                  gsem.at[6]).wait()

        def _shr(sc, ccx):
            lb = sc * SH
            gb = rbase + lb

            @pl.when(sc == 0)
            def _():
                pltpu.make_async_copy(h_hbm.at[pl.ds(rbase, SH)], hbuf_v,
                                      gsem.at[4]).wait()
                for q in range(NQ):
                    pltpu.make_async_copy(u_hbm[q].at[pl.ds(rbase, SH)],
                                          ub_v[q], gsem.at[q]).wait()

            @pl.when(sc > 0)
            def _():
                pltpu.sync_copy(h_hbm.at[pl.ds(gb, SH)], hbuf_v)
                for q in range(NQ):
                    pltpu.sync_copy(u_hbm[q].at[pl.ds(gb, SH)], ub_v[q])

            def _row(r, c3):
                rr = lb + r
                dv = dinv_v[rr, :]
                ds_, hs_ = [], []
                acc = None
                for q in range(NQ):
                    yq = dv * (Sq_v[q][rr, :] + ub_v[q][r, :])
                    hq = hbuf_v[r, pl.ds(q * QF, QF)]
                    dq = yq - hq
                    ds_.append(dq)
                    hs_.append(hq)
                    pq = dq * dq
                    acc = pq if acc is None else acc + pq
                total = jnp.sum(acc)
                rv = jnp.full((QF,), total)
                ri = _rsqrt16(rv)
                score = jnp.maximum(1.0 - 0.5 * ri, 0.0)
                for q in range(NQ):
                    xq = hs_[q] + score * ds_[q]
                    xkb_v[r, pl.ds(q * QF, QF)] = xq
                    ub_v[q][r, :] = xq * dv
                return c3
            lax.fori_loop(0, SH, _row, 0)

            @pl.when(k == K - 1)
            def _():
                pltpu.sync_copy(xkb_v, out_hbm.at[pl.ds(gb, SH)])
            for q in range(NQ):
                pltpu.sync_copy(ub_v[q], u_hbm[q].at[pl.ds(gb, SH)])
            return ccx
        lax.fori_loop(0, NSH, _shr, 0)
        plsc.subcore_barrier()
        _flag_write(2 + cc_, k + 2)
        return c
    lax.fori_loop(0, K, _round, 0)


def _propagate(h, rowi, coli, flg):
    mesh = plsc.VectorSubcoreMesh(
        core_axis_name="c", subcore_axis_name="s", num_cores=NCORE)
    f = pl.kernel(
        _sc_body,
        out_type=[jax.ShapeDtypeStruct((NPAD, F), jnp.float32)]
        + [jax.ShapeDtypeStruct((NPAD, QF), jnp.float32)] * (2 * NQ),
        mesh=mesh,
        compiler_params=pltpu.CompilerParams(
            needs_layout_passes=False, use_tc_tiling_on_sc=False),
        scratch_types=[
            pltpu.VMEM_SHARED((NPAD, QF), jnp.float32),
            pltpu.VMEM((NCH, C), jnp.int32),
            pltpu.VMEM((NCH, C), jnp.int32),
            pltpu.VMEM((RING, C, QF), jnp.float32),
            pltpu.VMEM((C, QF), jnp.float32),
            pltpu.VMEM((RPT, QF), jnp.float32),
            pltpu.VMEM((RPT, QF), jnp.float32),
            pltpu.VMEM((RPT, QF), jnp.float32),
            pltpu.VMEM((RPT, QF), jnp.float32),
            pltpu.VMEM((RPT, 16), jnp.float32),
            pltpu.VMEM((2, 16), jnp.int32),
            pltpu.VMEM((SH, F), jnp.float32),
            pltpu.VMEM((SH, F), jnp.float32),
            pltpu.VMEM((SH, QF), jnp.float32),
            pltpu.VMEM((SH, QF), jnp.float32),
            pltpu.VMEM((SH, QF), jnp.float32),
            pltpu.VMEM((SH, QF), jnp.float32),
            pltpu.SemaphoreType.DMA((RING,)),
            pltpu.SemaphoreType.DMA((RING,)),
        ],
    )
    return f(h, rowi, coli, flg)[0]


def kernel(x, edge_index, W1, b1, W2, b2):
    h = _mlp(x, W1, b1, W2, b2)

    row = edge_index[0].reshape(NSC, E // NSC)
    col = edge_index[1].reshape(NSC, E // NSC)
    pad = jnp.full((NSC, EPT_PAD - E // NSC), N, dtype=jnp.int32)
    rowi = jnp.concatenate([row, pad], axis=1).reshape(NSC, NCH, C)
    coli = jnp.concatenate([col, pad], axis=1).reshape(NSC, NCH, C)

    flg = jnp.zeros((4, 16), jnp.int32)
    xk = _propagate(h, rowi, coli, flg)
    return _log_softmax(xk)


# speedup vs baseline: 1.0080x; 1.0080x over previous
import jax
import jax.numpy as jnp
from jax import lax
from jax.experimental import pallas as pl
from jax.experimental.pallas import tpu as pltpu
from jax.experimental.pallas import tpu_sc as plsc

N = 10000
F = 64
E = 320000
K = 10
NSC = 16
NCORE = 2
NW = NSC * NCORE
NPAD = 10240
RPT = NPAD // NW
HROWS = NPAD // NCORE
ZPT = NPAD // NSC
SH = 160
NSH = RPT // SH
C = 128
NCH = 160
EPT_PAD = NCH * C
QF = 16
NQ = F // QF
RING = 8
DIST = RING // 2
NG = NCH // RING


def _rsqrt16(a):
    i = lax.bitcast_convert_type(a, jnp.int32)
    i = jnp.int32(0x5F3759DF) - lax.shift_right_arithmetic(i, 1)
    y = lax.bitcast_convert_type(i, jnp.float32)
    for _ in range(4):
        y = y * (1.5 - 0.5 * a * y * y)
    return y


def _mlp_body(x_ref, w1_ref, b1_ref, w2_ref, b2_ref, o_ref):
    h = jnp.dot(x_ref[...], w1_ref[...], preferred_element_type=jnp.float32)
    h = jnp.maximum(h + b1_ref[...], 0.0)
    o = jnp.dot(h, w2_ref[...], preferred_element_type=jnp.float32)
    o_ref[...] = o + b2_ref[...]


def _mlp(x, W1, b1, W2, b2):
    BR = 1000
    return pl.pallas_call(
        _mlp_body,
        grid=(N // BR,),
        in_specs=[
            pl.BlockSpec((BR, 128), lambda i: (i, 0)),
            pl.BlockSpec((128, 256), lambda i: (0, 0)),
            pl.BlockSpec((1, 256), lambda i: (0, 0)),
            pl.BlockSpec((256, F), lambda i: (0, 0)),
            pl.BlockSpec((1, F), lambda i: (0, 0)),
        ],
        out_specs=pl.BlockSpec((BR, F), lambda i: (i, 0)),
        out_shape=jax.ShapeDtypeStruct((NPAD, F), jnp.float32),
    )(x, W1, b1.reshape(1, 256), W2, b2.reshape(1, F))


def _lsm_body(x_ref, o_ref):
    v = x_ref[...]
    m = jnp.max(v, axis=1, keepdims=True)
    e = jnp.exp(v - m)
    s = jnp.sum(e, axis=1, keepdims=True)
    o_ref[...] = v - m - jnp.log(s)


def _log_softmax(x):
    BR = 1000
    return pl.pallas_call(
        _lsm_body,
        grid=(N // BR,),
        in_specs=[pl.BlockSpec((BR, F), lambda i: (i, 0))],
        out_specs=pl.BlockSpec((BR, F), lambda i: (i, 0)),
        out_shape=jax.ShapeDtypeStruct((N, F), jnp.float32),
    )(x)


def _sc_body(h_hbm, rowi_hbm, coli_hbm, flg_hbm,
             out_hbm, u0_hbm, u1_hbm, u2_hbm, u3_hbm,
             sx0_hbm, sx1_hbm, sx2_hbm, sx3_hbm,
             S_sh, rowi_v, coli_v, g_v, z_v,
             S0_v, S1_v, S2_v, S3_v, dinv_v, fb_v,
             hbuf_v, xkb_v, ub0_v, ub1_v, ub2_v, ub3_v, gsem, ssem):
    u_hbm = [u0_hbm, u1_hbm, u2_hbm, u3_hbm]
    sx_hbm = [sx0_hbm, sx1_hbm, sx2_hbm, sx3_hbm]
    Sq_v = [S0_v, S1_v, S2_v, S3_v]
    ub_v = [ub0_v, ub1_v, ub2_v, ub3_v]
    cc_ = lax.axis_index("c")
    s_ = lax.axis_index("s")
    w = cc_ * NSC + s_
    rbase = w * RPT
    zb = s_ * ZPT
    pxbase = (1 - cc_) * HROWS + s_ * RPT

    pltpu.sync_copy(rowi_hbm.at[s_], rowi_v)
    pltpu.sync_copy(coli_hbm.at[s_], coli_v)

    zv = jnp.zeros((QF,), jnp.float32)
    ov = jnp.full((QF,), 1.0, jnp.float32)

    def _fill(i, c):
        g_v[0, i, :] = ov
        z_v[i, :] = zv
        return c
    lax.fori_loop(0, C, _fill, 0)

    def _zero_zone():
        for z in range(ZPT // C):
            pltpu.sync_copy(z_v, S_sh.at[pl.ds(zb + z * C, C)])

    def _zero_rows320(base):
        pltpu.sync_copy(z_v, S_sh.at[pl.ds(base, C)])
        pltpu.sync_copy(z_v, S_sh.at[pl.ds(base + C, C)])
        pltpu.sync_copy(z_v.at[pl.ds(0, 64)], S_sh.at[pl.ds(base + 2 * C, 64)])

    def _flag_write(slot, val):
        @pl.when(s_ == 0)
        def _():
            fb_v[0, :] = jnp.full((QF,), val, jnp.int32)
            pltpu.sync_copy(fb_v.at[0], flg_hbm.at[slot])

    def _flag_poll(slot, val):
        @pl.when(s_ == 0)
        def _():
            def cond(carry):
                return carry < val

            def body(carry):
                pltpu.sync_copy(flg_hbm.at[slot], fb_v.at[1])
                return jnp.min(fb_v[1, :])
            lax.while_loop(cond, body, jnp.int32(-2147483647))

    _zero_zone()
    plsc.subcore_barrier()

    def _dgrp(jg, c):
        j0 = jg * RING
        for b in range(RING):
            j = j0 + b

            @pl.when(j >= RING)
            def _():
                pltpu.make_async_copy(
                    g_v.at[0], S_sh.at[rowi_v.at[j - RING]],
                    ssem.at[b]).wait()
            pltpu.async_copy(
                g_v.at[0], S_sh.at[rowi_v.at[j]], ssem.at[b], add=True)
        return c
    lax.fori_loop(0, NG, _dgrp, 0)
    for b in range(RING):
        j = NCH - RING + b
        pltpu.make_async_copy(
            g_v.at[0], S_sh.at[rowi_v.at[j]], ssem.at[b]).wait()
    plsc.subcore_barrier()

    pltpu.sync_copy(S_sh.at[pl.ds(rbase, RPT)], S0_v)
    plsc.subcore_barrier()
    _zero_zone()

    def _dinv(r, c):
        v = S0_v[r, :] + 1.0
        dinv_v[r, :] = _rsqrt16(v)
        return c
    lax.fori_loop(0, RPT, _dinv, 0)
    plsc.subcore_barrier()

    def _uinit(sc, c):
        gb = rbase + sc * SH
        pltpu.sync_copy(h_hbm.at[pl.ds(gb, SH)], hbuf_v)

        def _row(r, ccx):
            dv = dinv_v[sc * SH + r, :]
            for q in range(NQ):
                ub_v[q][r, :] = hbuf_v[r, pl.ds(q * QF, QF)] * dv
            return ccx
        lax.fori_loop(0, SH, _row, 0)
        for q in range(NQ):
            pltpu.sync_copy(ub_v[q], u_hbm[q].at[pl.ds(gb, SH)])
        return c
    lax.fori_loop(0, NSH, _uinit, 0)
    plsc.subcore_barrier()
    _flag_write(2 + cc_, jnp.int32(1))

    def _prime(u_ref):
        for b in range(DIST):
            pltpu.async_copy(u_ref.at[coli_v.at[b]], g_v.at[b], gsem.at[b])

    def _quarter_pass(u_ref):
        def _grp(jg, c):
            j0 = jg * RING
            for b in range(RING):
                j = j0 + b
                pltpu.make_async_copy(
                    u_ref.at[coli_v.at[j]], g_v.at[b], gsem.at[b]).wait()
                pltpu.async_copy(
                    g_v.at[b], S_sh.at[rowi_v.at[j]], ssem.at[b], add=True)
                bp = (b + DIST) % RING
                jold = j - DIST
                jp = j + DIST

                @pl.when(jold >= 0)
                def _():
                    pltpu.make_async_copy(
                        g_v.at[bp], S_sh.at[rowi_v.at[jold]],
                        ssem.at[bp]).wait()

                @pl.when(jp < NCH)
                def _():
                    pltpu.async_copy(
                        u_ref.at[coli_v.at[jp]], g_v.at[bp], gsem.at[bp])
            return c
        lax.fori_loop(0, NG, _grp, 0)
        for i in range(DIST):
            j = NCH - DIST + i
            b = j % RING
            pltpu.make_async_copy(
                g_v.at[b], S_sh.at[rowi_v.at[j]], ssem.at[b]).wait()
        plsc.subcore_barrier()

    def _export(q):
        pltpu.sync_copy(S_sh.at[pl.ds(rbase, RPT)], Sq_v[q])
        pltpu.sync_copy(S_sh.at[pl.ds(pxbase, RPT)],
                        sx_hbm[q].at[pl.ds(pxbase, RPT)])
        _zero_rows320(rbase)
        _zero_rows320(pxbase)

    def _two_passes(qa, qb):
        _prime(u_hbm[qa])
        _quarter_pass(u_hbm[qa])
        _prime(u_hbm[qb])
        _export(qa)
        plsc.subcore_barrier()
        _quarter_pass(u_hbm[qb])
        _export(qb)
        plsc.subcore_barrier()

    def _round(k, c):
        _flag_poll(3 - cc_, k + 1)
        plsc.subcore_barrier()

        @pl.when(cc_ == 0)
        def _():
            _two_passes(0, 1)

        @pl.when(cc_ == 1)
        def _():
            _two_passes(2, 3)

        pltpu.async_copy(h_hbm.at[pl.ds(rbase, SH)], hbuf_v, gsem.at[4])
        for q in range(NQ):
            pltpu.async_copy(u_hbm[q].at[pl.ds(rbase, SH)], ub_v[q],
                             gsem.at[q])

        _flag_write(cc_, k + 1)
        _flag_poll(1 - cc_, k + 1)
        plsc.subcore_barrier()

        @pl.when(cc_ == 0)
        def _():
            pltpu.async_copy(sx_hbm[2].at[pl.ds(rbase, RPT)], Sq_v[2],
                             gsem.at[5])
            pltpu.async_copy(sx_hbm[3].at[pl.ds(rbase, RPT)], Sq_v[3],
                             gsem.at[6])
            pltpu.make_async_copy(sx_hbm[2].at[pl.ds(rbase, RPT)], Sq_v[2],
                                  gsem.at[5]).wait()
            pltpu.make_async_copy(sx_hbm[3].at[pl.ds(rbase, RPT)], Sq_v[3],
                                  gsem.at[6]).wait()

        @pl.when(cc_ == 1)
        def _():
            pltpu.async_copy(sx_hbm[0].at[pl.ds(rbase, RPT)], Sq_v[0],
                             gsem.at[5])
            pltpu.async_copy(sx_hbm[1].at[pl.ds(rbase, RPT)], Sq_v[1],
                             gsem.at[6])
            pltpu.make_async_copy(sx_hbm[0].at[pl.ds(rbase, RPT)], Sq_v[0],
                                  gsem.at[5]).wait()
            pltpu.make_async_copy(sx_hbm[1].at[pl.ds(rbase, RPT)], Sq_v[1],
                                  gsem.at[6]).wait()

        def _shr(sc, ccx):
            lb = sc * SH
            gb = rbase + lb

            @pl.when(sc == 0)
            def _():
                pltpu.make_async_copy(h_hbm.at[pl.ds(rbase, SH)], hbuf_v,
                                      gsem.at[4]).wait()
                for q in range(NQ):
                    pltpu.make_async_copy(u_hbm[q].at[pl.ds(rbase, SH)],
                                          ub_v[q], gsem.at[q]).wait()

            @pl.when(sc > 0)
            def _():
                pltpu.sync_copy(h_hbm.at[pl.ds(gb, SH)], hbuf_v)
                for q in range(NQ):
                    pltpu.sync_copy(u_hbm[q].at[pl.ds(gb, SH)], ub_v[q])

            def _row(r, c3):
                rr = lb + r
                dv = dinv_v[rr, :]
                ds_, hs_ = [], []
                acc = None
                for q in range(NQ):
                    yq = dv * (Sq_v[q][rr, :] + ub_v[q][r, :])
                    hq = hbuf_v[r, pl.ds(q * QF, QF)]
                    dq = yq - hq
                    ds_.append(dq)
                    hs_.append(hq)
                    pq = dq * dq
                    acc = pq if acc is None else acc + pq
                total = jnp.sum(acc)
                rv = jnp.full((QF,), total)
                ri = _rsqrt16(rv)
                score = jnp.maximum(1.0 - 0.5 * ri, 0.0)
                for q in range(NQ):
                    xq = hs_[q] + score * ds_[q]
                    xkb_v[r, pl.ds(q * QF, QF)] = xq
                    ub_v[q][r, :] = xq * dv
                return c3
            lax.fori_loop(0, SH, _row, 0)

            @pl.when(k == K - 1)
            def _():
                pltpu.sync_copy(xkb_v, out_hbm.at[pl.ds(gb, SH)])
            for q in range(NQ):
                pltpu.sync_copy(ub_v[q], u_hbm[q].at[pl.ds(gb, SH)])
            return ccx
        lax.fori_loop(0, NSH, _shr, 0)
        plsc.subcore_barrier()
        _flag_write(2 + cc_, k + 2)
        return c
    lax.fori_loop(0, K, _round, 0)


def _propagate(h, rowi, coli, flg):
    mesh = plsc.VectorSubcoreMesh(
        core_axis_name="c", subcore_axis_name="s", num_cores=NCORE)
    f = pl.kernel(
        _sc_body,
        out_type=[jax.ShapeDtypeStruct((NPAD, F), jnp.float32)]
        + [jax.ShapeDtypeStruct((NPAD, QF), jnp.float32)] * (2 * NQ),
        mesh=mesh,
        compiler_params=pltpu.CompilerParams(
            needs_layout_passes=False, use_tc_tiling_on_sc=False),
        scratch_types=[
            pltpu.VMEM_SHARED((NPAD, QF), jnp.float32),
            pltpu.VMEM((NCH, C), jnp.int32),
            pltpu.VMEM((NCH, C), jnp.int32),
            pltpu.VMEM((RING, C, QF), jnp.float32),
            pltpu.VMEM((C, QF), jnp.float32),
            pltpu.VMEM((RPT, QF), jnp.float32),
            pltpu.VMEM((RPT, QF), jnp.float32),
            pltpu.VMEM((RPT, QF), jnp.float32),
            pltpu.VMEM((RPT, QF), jnp.float32),
            pltpu.VMEM((RPT, 16), jnp.float32),
            pltpu.VMEM((2, 16), jnp.int32),
            pltpu.VMEM((SH, F), jnp.float32),
            pltpu.VMEM((SH, F), jnp.float32),
            pltpu.VMEM((SH, QF), jnp.float32),
            pltpu.VMEM((SH, QF), jnp.float32),
            pltpu.VMEM((SH, QF), jnp.float32),
            pltpu.VMEM((SH, QF), jnp.float32),
            pltpu.SemaphoreType.DMA((RING,)),
            pltpu.SemaphoreType.DMA((RING,)),
        ],
    )
    return f(h, rowi, coli, flg)[0]


def kernel(x, edge_index, W1, b1, W2, b2):
    h = _mlp(x, W1, b1, W2, b2)

    row = edge_index[0].reshape(NSC, E // NSC)
    col = edge_index[1].reshape(NSC, E // NSC)
    pad = jnp.full((NSC, EPT_PAD - E // NSC), N, dtype=jnp.int32)
    rowi = jnp.concatenate([row, pad], axis=1).reshape(NSC, NCH, C)
    coli = jnp.concatenate([col, pad], axis=1).reshape(NSC, NCH, C)

    flg = jnp.zeros((4, 16), jnp.int32)
    xk = _propagate(h, rowi, coli, flg)
    return _log_softmax(xk)


# speedup vs baseline: 1.0697x; 1.0613x over previous
import jax
import jax.numpy as jnp
from jax import lax
from jax.experimental import pallas as pl
from jax.experimental.pallas import tpu as pltpu
from jax.experimental.pallas import tpu_sc as plsc

N = 10000
F = 64
E = 320000
K = 10
NSC = 16
NCORE = 2
NW = NSC * NCORE
NPAD = 10240
RPT = NPAD // NW
HROWS = NPAD // NCORE
ZPT = NPAD // NSC
SH = 160
NSH = RPT // SH
C = 128
NCH = 160
EPT_PAD = NCH * C
QF = 16
NQ = F // QF
RING = 10
DIST = RING // 2
NG = NCH // RING


def _rsqrt16(a):
    i = lax.bitcast_convert_type(a, jnp.int32)
    i = jnp.int32(0x5F3759DF) - lax.shift_right_arithmetic(i, 1)
    y = lax.bitcast_convert_type(i, jnp.float32)
    for _ in range(4):
        y = y * (1.5 - 0.5 * a * y * y)
    return y


def _mlp_body(x_ref, w1_ref, b1_ref, w2_ref, b2_ref, o_ref):
    h = jnp.dot(x_ref[...], w1_ref[...], preferred_element_type=jnp.float32)
    h = jnp.maximum(h + b1_ref[...], 0.0)
    o = jnp.dot(h, w2_ref[...], preferred_element_type=jnp.float32)
    o_ref[...] = o + b2_ref[...]


def _mlp(x, W1, b1, W2, b2):
    BR = 1000
    return pl.pallas_call(
        _mlp_body,
        grid=(N // BR,),
        in_specs=[
            pl.BlockSpec((BR, 128), lambda i: (i, 0)),
            pl.BlockSpec((128, 256), lambda i: (0, 0)),
            pl.BlockSpec((1, 256), lambda i: (0, 0)),
            pl.BlockSpec((256, F), lambda i: (0, 0)),
            pl.BlockSpec((1, F), lambda i: (0, 0)),
        ],
        out_specs=pl.BlockSpec((BR, F), lambda i: (i, 0)),
        out_shape=jax.ShapeDtypeStruct((NPAD, F), jnp.float32),
    )(x, W1, b1.reshape(1, 256), W2, b2.reshape(1, F))


def _lsm_body(x_ref, o_ref):
    v = x_ref[...]
    m = jnp.max(v, axis=1, keepdims=True)
    e = jnp.exp(v - m)
    s = jnp.sum(e, axis=1, keepdims=True)
    o_ref[...] = v - m - jnp.log(s)


def _log_softmax(x):
    BR = 1000
    return pl.pallas_call(
        _lsm_body,
        grid=(N // BR,),
        in_specs=[pl.BlockSpec((BR, F), lambda i: (i, 0))],
        out_specs=pl.BlockSpec((BR, F), lambda i: (i, 0)),
        out_shape=jax.ShapeDtypeStruct((N, F), jnp.float32),
    )(x)


def _sc_body(h_hbm, rowi_hbm, coli_hbm, flg_hbm,
             out_hbm, u0_hbm, u1_hbm, u2_hbm, u3_hbm,
             sx0_hbm, sx1_hbm, sx2_hbm, sx3_hbm,
             S_sh, rowi_v, coli_v, g_v, z_v,
             S0_v, S1_v, S2_v, S3_v, dinv_v, fb_v,
             hbuf_v, xkb_v, ub0_v, ub1_v, ub2_v, ub3_v, gsem, ssem):
    u_hbm = [u0_hbm, u1_hbm, u2_hbm, u3_hbm]
    sx_hbm = [sx0_hbm, sx1_hbm, sx2_hbm, sx3_hbm]
    Sq_v = [S0_v, S1_v, S2_v, S3_v]
    ub_v = [ub0_v, ub1_v, ub2_v, ub3_v]
    cc_ = lax.axis_index("c")
    s_ = lax.axis_index("s")
    w = cc_ * NSC + s_
    rbase = w * RPT
    zb = s_ * ZPT
    pxbase = (1 - cc_) * HROWS + s_ * RPT

    pltpu.sync_copy(rowi_hbm.at[s_], rowi_v)
    pltpu.sync_copy(coli_hbm.at[s_], coli_v)

    zv = jnp.zeros((QF,), jnp.float32)
    ov = jnp.full((QF,), 1.0, jnp.float32)

    def _fill(i, c):
        g_v[0, i, :] = ov
        z_v[i, :] = zv
        return c
    lax.fori_loop(0, C, _fill, 0)

    def _zero_zone():
        for z in range(ZPT // C):
            pltpu.sync_copy(z_v, S_sh.at[pl.ds(zb + z * C, C)])

    def _zero_rows320(base):
        pltpu.sync_copy(z_v, S_sh.at[pl.ds(base, C)])
        pltpu.sync_copy(z_v, S_sh.at[pl.ds(base + C, C)])
        pltpu.sync_copy(z_v.at[pl.ds(0, 64)], S_sh.at[pl.ds(base + 2 * C, 64)])

    def _flag_write(slot, val):
        @pl.when(s_ == 0)
        def _():
            fb_v[0, :] = jnp.full((QF,), val, jnp.int32)
            pltpu.sync_copy(fb_v.at[0], flg_hbm.at[slot])

    def _flag_poll(slot, val):
        @pl.when(s_ == 0)
        def _():
            def cond(carry):
                return carry < val

            def body(carry):
                pltpu.sync_copy(flg_hbm.at[slot], fb_v.at[1])
                return jnp.min(fb_v[1, :])
            lax.while_loop(cond, body, jnp.int32(-2147483647))

    _zero_zone()
    plsc.subcore_barrier()

    def _dgrp(jg, c):
        j0 = jg * RING
        for b in range(RING):
            j = j0 + b

            @pl.when(j >= RING)
            def _():
                pltpu.make_async_copy(
                    g_v.at[0], S_sh.at[rowi_v.at[j - RING]],
                    ssem.at[b]).wait()
            pltpu.async_copy(
                g_v.at[0], S_sh.at[rowi_v.at[j]], ssem.at[b], add=True)
        return c
    lax.fori_loop(0, NG, _dgrp, 0)
    for b in range(RING):
        j = NCH - RING + b
        pltpu.make_async_copy(
            g_v.at[0], S_sh.at[rowi_v.at[j]], ssem.at[b]).wait()
    plsc.subcore_barrier()

    pltpu.sync_copy(S_sh.at[pl.ds(rbase, RPT)], S0_v)
    plsc.subcore_barrier()
    _zero_zone()

    def _dinv(r, c):
        v = S0_v[r, :] + 1.0
        dinv_v[r, :] = _rsqrt16(v)
        return c
    lax.fori_loop(0, RPT, _dinv, 0)
    plsc.subcore_barrier()

    def _uinit(sc, c):
        gb = rbase + sc * SH
        pltpu.sync_copy(h_hbm.at[pl.ds(gb, SH)], hbuf_v)

        def _row(r, ccx):
            dv = dinv_v[sc * SH + r, :]
            for q in range(NQ):
                ub_v[q][r, :] = hbuf_v[r, pl.ds(q * QF, QF)] * dv
            return ccx
        lax.fori_loop(0, SH, _row, 0)
        for q in range(NQ):
            pltpu.sync_copy(ub_v[q], u_hbm[q].at[pl.ds(gb, SH)])
        return c
    lax.fori_loop(0, NSH, _uinit, 0)
    plsc.subcore_barrier()
    _flag_write(2 + cc_, jnp.int32(1))

    def _prime(u_ref):
        for b in range(DIST):
            pltpu.async_copy(u_ref.at[coli_v.at[b]], g_v.at[b], gsem.at[b])

    def _quarter_pass(u_ref):
        def _grp(jg, c):
            j0 = jg * RING
            for b in range(RING):
                j = j0 + b
                pltpu.make_async_copy(
                    u_ref.at[coli_v.at[j]], g_v.at[b], gsem.at[b]).wait()
                pltpu.async_copy(
                    g_v.at[b], S_sh.at[rowi_v.at[j]], ssem.at[b], add=True)
                bp = (b + DIST) % RING
                jold = j - DIST
                jp = j + DIST

                @pl.when(jold >= 0)
                def _():
                    pltpu.make_async_copy(
                        g_v.at[bp], S_sh.at[rowi_v.at[jold]],
                        ssem.at[bp]).wait()

                @pl.when(jp < NCH)
                def _():
                    pltpu.async_copy(
                        u_ref.at[coli_v.at[jp]], g_v.at[bp], gsem.at[bp])
            return c
        lax.fori_loop(0, NG, _grp, 0)
        for i in range(DIST):
            j = NCH - DIST + i
            b = j % RING
            pltpu.make_async_copy(
                g_v.at[b], S_sh.at[rowi_v.at[j]], ssem.at[b]).wait()
        plsc.subcore_barrier()

    def _export(q):
        pltpu.sync_copy(S_sh.at[pl.ds(rbase, RPT)], Sq_v[q])
        pltpu.sync_copy(S_sh.at[pl.ds(pxbase, RPT)],
                        sx_hbm[q].at[pl.ds(pxbase, RPT)])
        _zero_rows320(rbase)
        _zero_rows320(pxbase)

    def _two_passes(qa, qb):
        _prime(u_hbm[qa])
        _quarter_pass(u_hbm[qa])
        _prime(u_hbm[qb])
        _export(qa)
        plsc.subcore_barrier()
        _quarter_pass(u_hbm[qb])
        _export(qb)
        plsc.subcore_barrier()

    def _round(k, c):
        _flag_poll(3 - cc_, k + 1)
        plsc.subcore_barrier()

        @pl.when(cc_ == 0)
        def _():
            _two_passes(0, 1)

        @pl.when(cc_ == 1)
        def _():
            _two_passes(2, 3)

        pltpu.async_copy(h_hbm.at[pl.ds(rbase, SH)], hbuf_v, gsem.at[4])
        for q in range(NQ):
            pltpu.async_copy(u_hbm[q].at[pl.ds(rbase, SH)], ub_v[q],
                             gsem.at[q])

        _flag_write(cc_, k + 1)
        _flag_poll(1 - cc_, k + 1)
        plsc.subcore_barrier()

        @pl.when(cc_ == 0)
        def _():
            pltpu.async_copy(sx_hbm[2].at[pl.ds(rbase, RPT)], Sq_v[2],
                             gsem.at[5])
            pltpu.async_copy(sx_hbm[3].at[pl.ds(rbase, RPT)], Sq_v[3],
                             gsem.at[6])
            pltpu.make_async_copy(sx_hbm[2].at[pl.ds(rbase, RPT)], Sq_v[2],
                                  gsem.at[5]).wait()
            pltpu.make_async_copy(sx_hbm[3].at[pl.ds(rbase, RPT)], Sq_v[3],
                                  gsem.at[6]).wait()

        @pl.when(cc_ == 1)
        def _():
            pltpu.async_copy(sx_hbm[0].at[pl.ds(rbase, RPT)], Sq_v[0],
                             gsem.at[5])
            pltpu.async_copy(sx_hbm[1].at[pl.ds(rbase, RPT)], Sq_v[1],
                             gsem.at[6])
            pltpu.make_async_copy(sx_hbm[0].at[pl.ds(rbase, RPT)], Sq_v[0],
                                  gsem.at[5]).wait()
            pltpu.make_async_copy(sx_hbm[1].at[pl.ds(rbase, RPT)], Sq_v[1],
                                  gsem.at[6]).wait()

        def _shr(sc, ccx):
            lb = sc * SH
            gb = rbase + lb

            @pl.when(sc == 0)
            def _():
                pltpu.make_async_copy(h_hbm.at[pl.ds(rbase, SH)], hbuf_v,
                                      gsem.at[4]).wait()
                for q in range(NQ):
                    pltpu.make_async_copy(u_hbm[q].at[pl.ds(rbase, SH)],
                                          ub_v[q], gsem.at[q]).wait()

            @pl.when(sc > 0)
            def _():
                pltpu.sync_copy(h_hbm.at[pl.ds(gb, SH)], hbuf_v)
                for q in range(NQ):
                    pltpu.sync_copy(u_hbm[q].at[pl.ds(gb, SH)], ub_v[q])

            def _row(r, c3):
                rr = lb + r
                dv = dinv_v[rr, :]
                ds_, hs_ = [], []
                acc = None
                for q in range(NQ):
                    yq = dv * (Sq_v[q][rr, :] + ub_v[q][r, :])
                    hq = hbuf_v[r, pl.ds(q * QF, QF)]
                    dq = yq - hq
                    ds_.append(dq)
                    hs_.append(hq)
                    pq = dq * dq
                    acc = pq if acc is None else acc + pq
                total = jnp.sum(acc)
                rv = jnp.full((QF,), total)
                ri = _rsqrt16(rv)
                score = jnp.maximum(1.0 - 0.5 * ri, 0.0)
                for q in range(NQ):
                    xq = hs_[q] + score * ds_[q]
                    xkb_v[r, pl.ds(q * QF, QF)] = xq
                    ub_v[q][r, :] = xq * dv
                return c3
            lax.fori_loop(0, SH, _row, 0)

            @pl.when(k == K - 1)
            def _():
                pltpu.sync_copy(xkb_v, out_hbm.at[pl.ds(gb, SH)])
            for q in range(NQ):
                pltpu.sync_copy(ub_v[q], u_hbm[q].at[pl.ds(gb, SH)])
            return ccx
        lax.fori_loop(0, NSH, _shr, 0)
        plsc.subcore_barrier()
        _flag_write(2 + cc_, k + 2)
        return c
    lax.fori_loop(0, K, _round, 0)


def _propagate(h, rowi, coli, flg):
    mesh = plsc.VectorSubcoreMesh(
        core_axis_name="c", subcore_axis_name="s", num_cores=NCORE)
    f = pl.kernel(
        _sc_body,
        out_type=[jax.ShapeDtypeStruct((NPAD, F), jnp.float32)]
        + [jax.ShapeDtypeStruct((NPAD, QF), jnp.float32)] * (2 * NQ),
        mesh=mesh,
        compiler_params=pltpu.CompilerParams(
            needs_layout_passes=False, use_tc_tiling_on_sc=False),
        scratch_types=[
            pltpu.VMEM_SHARED((NPAD, QF), jnp.float32),
            pltpu.VMEM((NCH, C), jnp.int32),
            pltpu.VMEM((NCH, C), jnp.int32),
            pltpu.VMEM((RING, C, QF), jnp.float32),
            pltpu.VMEM((C, QF), jnp.float32),
            pltpu.VMEM((RPT, QF), jnp.float32),
            pltpu.VMEM((RPT, QF), jnp.float32),
            pltpu.VMEM((RPT, QF), jnp.float32),
            pltpu.VMEM((RPT, QF), jnp.float32),
            pltpu.VMEM((RPT, 16), jnp.float32),
            pltpu.VMEM((2, 16), jnp.int32),
            pltpu.VMEM((SH, F), jnp.float32),
            pltpu.VMEM((SH, F), jnp.float32),
            pltpu.VMEM((SH, QF), jnp.float32),
            pltpu.VMEM((SH, QF), jnp.float32),
            pltpu.VMEM((SH, QF), jnp.float32),
            pltpu.VMEM((SH, QF), jnp.float32),
            pltpu.SemaphoreType.DMA((RING,)),
            pltpu.SemaphoreType.DMA((RING,)),
        ],
    )
    return f(h, rowi, coli, flg)[0]


def kernel(x, edge_index, W1, b1, W2, b2):
    h = _mlp(x, W1, b1, W2, b2)

    row = edge_index[0].reshape(NSC, E // NSC)
    col = edge_index[1].reshape(NSC, E // NSC)
    pad = jnp.full((NSC, EPT_PAD - E // NSC), N, dtype=jnp.int32)
    rowi = jnp.concatenate([row, pad], axis=1).reshape(NSC, NCH, C)
    coli = jnp.concatenate([col, pad], axis=1).reshape(NSC, NCH, C)

    flg = jnp.zeros((4, 16), jnp.int32)
    xk = _propagate(h, rowi, coli, flg)
    return _log_softmax(xk)


# speedup vs baseline: 1.0803x; 1.0099x over previous
import jax
import jax.numpy as jnp
from jax import lax
from jax.experimental import pallas as pl
from jax.experimental.pallas import tpu as pltpu
from jax.experimental.pallas import tpu_sc as plsc

N = 10000
F = 64
E = 320000
K = 10
NSC = 16
NCORE = 2
NW = NSC * NCORE
NPAD = 10240
RPT = NPAD // NW
HROWS = NPAD // NCORE
ZPT = NPAD // NSC
SH = 160
NSH = RPT // SH
C = 128
NCH = 160
EPT_PAD = NCH * C
QF = 16
NQ = F // QF
RING = 10
DIST = RING // 2
NG = NCH // RING


def _rsqrt16(a):
    i = lax.bitcast_convert_type(a, jnp.int32)
    i = jnp.int32(0x5F3759DF) - lax.shift_right_arithmetic(i, 1)
    y = lax.bitcast_convert_type(i, jnp.float32)
    for _ in range(3):
        y = y * (1.5 - 0.5 * a * y * y)
    return y


def _mlp_body(x_ref, w1_ref, b1_ref, w2_ref, b2_ref, o_ref):
    h = jnp.dot(x_ref[...], w1_ref[...], preferred_element_type=jnp.float32)
    h = jnp.maximum(h + b1_ref[...], 0.0)
    o = jnp.dot(h, w2_ref[...], preferred_element_type=jnp.float32)
    o_ref[...] = o + b2_ref[...]


def _mlp(x, W1, b1, W2, b2):
    BR = 1000
    return pl.pallas_call(
        _mlp_body,
        grid=(N // BR,),
        in_specs=[
            pl.BlockSpec((BR, 128), lambda i: (i, 0)),
            pl.BlockSpec((128, 256), lambda i: (0, 0)),
            pl.BlockSpec((1, 256), lambda i: (0, 0)),
            pl.BlockSpec((256, F), lambda i: (0, 0)),
            pl.BlockSpec((1, F), lambda i: (0, 0)),
        ],
        out_specs=pl.BlockSpec((BR, F), lambda i: (i, 0)),
        out_shape=jax.ShapeDtypeStruct((NPAD, F), jnp.float32),
    )(x, W1, b1.reshape(1, 256), W2, b2.reshape(1, F))


def _lsm_body(x_ref, o_ref):
    v = x_ref[...]
    m = jnp.max(v, axis=1, keepdims=True)
    e = jnp.exp(v - m)
    s = jnp.sum(e, axis=1, keepdims=True)
    o_ref[...] = v - m - jnp.log(s)


def _log_softmax(x):
    BR = 1000
    return pl.pallas_call(
        _lsm_body,
        grid=(N // BR,),
        in_specs=[pl.BlockSpec((BR, F), lambda i: (i, 0))],
        out_specs=pl.BlockSpec((BR, F), lambda i: (i, 0)),
        out_shape=jax.ShapeDtypeStruct((N, F), jnp.float32),
    )(x)


def _sc_body(h_hbm, rowi_hbm, coli_hbm, flg_hbm,
             out_hbm, u0_hbm, u1_hbm, u2_hbm, u3_hbm,
             sx0_hbm, sx1_hbm, sx2_hbm, sx3_hbm,
             S_sh, rowi_v, coli_v, g_v, z_v,
             S0_v, S1_v, S2_v, S3_v, dinv_v, fb_v,
             hbuf_v, xkb_v, ub0_v, ub1_v, ub2_v, ub3_v, gsem, ssem):
    u_hbm = [u0_hbm, u1_hbm, u2_hbm, u3_hbm]
    sx_hbm = [sx0_hbm, sx1_hbm, sx2_hbm, sx3_hbm]
    Sq_v = [S0_v, S1_v, S2_v, S3_v]
    ub_v = [ub0_v, ub1_v, ub2_v, ub3_v]
    cc_ = lax.axis_index("c")
    s_ = lax.axis_index("s")
    w = cc_ * NSC + s_
    rbase = w * RPT
    zb = s_ * ZPT
    pxbase = (1 - cc_) * HROWS + s_ * RPT

    pltpu.sync_copy(rowi_hbm.at[s_], rowi_v)
    pltpu.sync_copy(coli_hbm.at[s_], coli_v)

    zv = jnp.zeros((QF,), jnp.float32)
    ov = jnp.full((QF,), 1.0, jnp.float32)

    def _fill(i, c):
        g_v[0, i, :] = ov
        z_v[i, :] = zv
        return c
    lax.fori_loop(0, C, _fill, 0)

    def _zero_zone():
        for z in range(ZPT // C):
            pltpu.sync_copy(z_v, S_sh.at[pl.ds(zb + z * C, C)])

    def _zero_rows320(base):
        pltpu.sync_copy(z_v, S_sh.at[pl.ds(base, C)])
        pltpu.sync_copy(z_v, S_sh.at[pl.ds(base + C, C)])
        pltpu.sync_copy(z_v.at[pl.ds(0, 64)], S_sh.at[pl.ds(base + 2 * C, 64)])

    def _flag_write(slot, val):
        @pl.when(s_ == 0)
        def _():
            fb_v[0, :] = jnp.full((QF,), val, jnp.int32)
            pltpu.sync_copy(fb_v.at[0], flg_hbm.at[slot])

    def _flag_poll(slot, val):
        @pl.when(s_ == 0)
        def _():
            def cond(carry):
                return carry < val

            def body(carry):
                pltpu.sync_copy(flg_hbm.at[slot], fb_v.at[1])
                return jnp.min(fb_v[1, :])
            lax.while_loop(cond, body, jnp.int32(-2147483647))

    _zero_zone()
    plsc.subcore_barrier()

    def _dgrp(jg, c):
        j0 = jg * RING
        for b in range(RING):
            j = j0 + b

            @pl.when(j >= RING)
            def _():
                pltpu.make_async_copy(
                    g_v.at[0], S_sh.at[rowi_v.at[j - RING]],
                    ssem.at[b]).wait()
            pltpu.async_copy(
                g_v.at[0], S_sh.at[rowi_v.at[j]], ssem.at[b], add=True)
        return c
    lax.fori_loop(0, NG, _dgrp, 0)
    for b in range(RING):
        j = NCH - RING + b
        pltpu.make_async_copy(
            g_v.at[0], S_sh.at[rowi_v.at[j]], ssem.at[b]).wait()
    plsc.subcore_barrier()

    pltpu.sync_copy(S_sh.at[pl.ds(rbase, RPT)], S0_v)
    plsc.subcore_barrier()
    _zero_zone()

    def _dinv(r, c):
        v = S0_v[r, :] + 1.0
        dinv_v[r, :] = _rsqrt16(v)
        return c
    lax.fori_loop(0, RPT, _dinv, 0)
    plsc.subcore_barrier()

    def _uinit(sc, c):
        gb = rbase + sc * SH
        pltpu.sync_copy(h_hbm.at[pl.ds(gb, SH)], hbuf_v)

        def _row(r, ccx):
            dv = dinv_v[sc * SH + r, :]
            for q in range(NQ):
                ub_v[q][r, :] = hbuf_v[r, pl.ds(q * QF, QF)] * dv
            return ccx
        lax.fori_loop(0, SH, _row, 0)
        for q in range(NQ):
            pltpu.sync_copy(ub_v[q], u_hbm[q].at[pl.ds(gb, SH)])
        return c
    lax.fori_loop(0, NSH, _uinit, 0)
    plsc.subcore_barrier()
    _flag_write(2 + cc_, jnp.int32(1))

    def _prime(u_ref):
        for b in range(DIST):
            pltpu.async_copy(u_ref.at[coli_v.at[b]], g_v.at[b], gsem.at[b])

    def _quarter_pass(u_ref):
        def _grp(jg, c):
            j0 = jg * RING
            for b in range(RING):
                j = j0 + b
                pltpu.make_async_copy(
                    u_ref.at[coli_v.at[j]], g_v.at[b], gsem.at[b]).wait()
                pltpu.async_copy(
                    g_v.at[b], S_sh.at[rowi_v.at[j]], ssem.at[b], add=True)
                bp = (b + DIST) % RING
                jold = j - DIST
                jp = j + DIST

                @pl.when(jold >= 0)
                def _():
                    pltpu.make_async_copy(
                        g_v.at[bp], S_sh.at[rowi_v.at[jold]],
                        ssem.at[bp]).wait()

                @pl.when(jp < NCH)
                def _():
                    pltpu.async_copy(
                        u_ref.at[coli_v.at[jp]], g_v.at[bp], gsem.at[bp])
            return c
        lax.fori_loop(0, NG, _grp, 0)
        for i in range(DIST):
            j = NCH - DIST + i
            b = j % RING
            pltpu.make_async_copy(
                g_v.at[b], S_sh.at[rowi_v.at[j]], ssem.at[b]).wait()
        plsc.subcore_barrier()

    def _export(q):
        pltpu.sync_copy(S_sh.at[pl.ds(rbase, RPT)], Sq_v[q])
        pltpu.sync_copy(S_sh.at[pl.ds(pxbase, RPT)],
                        sx_hbm[q].at[pl.ds(pxbase, RPT)])
        _zero_rows320(rbase)
        _zero_rows320(pxbase)

    def _two_passes(qa, qb):
        _prime(u_hbm[qa])
        _quarter_pass(u_hbm[qa])
        _prime(u_hbm[qb])
        _export(qa)
        plsc.subcore_barrier()
        _quarter_pass(u_hbm[qb])
        _export(qb)
        plsc.subcore_barrier()

    def _round(k, c):
        _flag_poll(3 - cc_, k + 1)
        plsc.subcore_barrier()

        @pl.when(cc_ == 0)
        def _():
            _two_passes(0, 1)

        @pl.when(cc_ == 1)
        def _():
            _two_passes(2, 3)

        pltpu.async_copy(h_hbm.at[pl.ds(rbase, SH)], hbuf_v, gsem.at[4])
        for q in range(NQ):
            pltpu.async_copy(u_hbm[q].at[pl.ds(rbase, SH)], ub_v[q],
                             gsem.at[q])

        _flag_write(cc_, k + 1)
        _flag_poll(1 - cc_, k + 1)
        plsc.subcore_barrier()

        @pl.when(cc_ == 0)
        def _():
            pltpu.async_copy(sx_hbm[2].at[pl.ds(rbase, RPT)], Sq_v[2],
                             gsem.at[5])
            pltpu.async_copy(sx_hbm[3].at[pl.ds(rbase, RPT)], Sq_v[3],
                             gsem.at[6])
            pltpu.make_async_copy(sx_hbm[2].at[pl.ds(rbase, RPT)], Sq_v[2],
                                  gsem.at[5]).wait()
            pltpu.make_async_copy(sx_hbm[3].at[pl.ds(rbase, RPT)], Sq_v[3],
                                  gsem.at[6]).wait()

        @pl.when(cc_ == 1)
        def _():
            pltpu.async_copy(sx_hbm[0].at[pl.ds(rbase, RPT)], Sq_v[0],
                             gsem.at[5])
            pltpu.async_copy(sx_hbm[1].at[pl.ds(rbase, RPT)], Sq_v[1],
                             gsem.at[6])
            pltpu.make_async_copy(sx_hbm[0].at[pl.ds(rbase, RPT)], Sq_v[0],
                                  gsem.at[5]).wait()
            pltpu.make_async_copy(sx_hbm[1].at[pl.ds(rbase, RPT)], Sq_v[1],
                                  gsem.at[6]).wait()

        def _shr(sc, ccx):
            lb = sc * SH
            gb = rbase + lb

            @pl.when(sc == 0)
            def _():
                pltpu.make_async_copy(h_hbm.at[pl.ds(rbase, SH)], hbuf_v,
                                      gsem.at[4]).wait()
                for q in range(NQ):
                    pltpu.make_async_copy(u_hbm[q].at[pl.ds(rbase, SH)],
                                          ub_v[q], gsem.at[q]).wait()

            @pl.when(sc > 0)
            def _():
                pltpu.sync_copy(h_hbm.at[pl.ds(gb, SH)], hbuf_v)
                for q in range(NQ):
                    pltpu.sync_copy(u_hbm[q].at[pl.ds(gb, SH)], ub_v[q])

            def _row2(r2, c3):
                for r in (2 * r2, 2 * r2 + 1):
                    rr = lb + r
                    dv = dinv_v[rr, :]
                    ds_, hs_ = [], []
                    acc = None
                    for q in range(NQ):
                        yq = dv * (Sq_v[q][rr, :] + ub_v[q][r, :])
                        hq = hbuf_v[r, pl.ds(q * QF, QF)]
                        dq = yq - hq
                        ds_.append(dq)
                        hs_.append(hq)
                        pq = dq * dq
                        acc = pq if acc is None else acc + pq
                    total = jnp.sum(acc)
                    rv = jnp.full((QF,), total)
                    ri = _rsqrt16(rv)
                    score = jnp.maximum(1.0 - 0.5 * ri, 0.0)
                    for q in range(NQ):
                        xq = hs_[q] + score * ds_[q]
                        xkb_v[r, pl.ds(q * QF, QF)] = xq
                        ub_v[q][r, :] = xq * dv
                return c3
            lax.fori_loop(0, SH // 2, _row2, 0)

            @pl.when(k == K - 1)
            def _():
                pltpu.sync_copy(xkb_v, out_hbm.at[pl.ds(gb, SH)])
            for q in range(NQ):
                pltpu.sync_copy(ub_v[q], u_hbm[q].at[pl.ds(gb, SH)])
            return ccx
        lax.fori_loop(0, NSH, _shr, 0)
        plsc.subcore_barrier()
        _flag_write(2 + cc_, k + 2)
        return c
    lax.fori_loop(0, K, _round, 0)


def _propagate(h, rowi, coli, flg):
    mesh = plsc.VectorSubcoreMesh(
        core_axis_name="c", subcore_axis_name="s", num_cores=NCORE)
    f = pl.kernel(
        _sc_body,
        out_type=[jax.ShapeDtypeStruct((NPAD, F), jnp.float32)]
        + [jax.ShapeDtypeStruct((NPAD, QF), jnp.float32)] * (2 * NQ),
        mesh=mesh,
        compiler_params=pltpu.CompilerParams(
            needs_layout_passes=False, use_tc_tiling_on_sc=False),
        scratch_types=[
            pltpu.VMEM_SHARED((NPAD, QF), jnp.float32),
            pltpu.VMEM((NCH, C), jnp.int32),
            pltpu.VMEM((NCH, C), jnp.int32),
            pltpu.VMEM((RING, C, QF), jnp.float32),
            pltpu.VMEM((C, QF), jnp.float32),
            pltpu.VMEM((RPT, QF), jnp.float32),
            pltpu.VMEM((RPT, QF), jnp.float32),
            pltpu.VMEM((RPT, QF), jnp.float32),
            pltpu.VMEM((RPT, QF), jnp.float32),
            pltpu.VMEM((RPT, 16), jnp.float32),
            pltpu.VMEM((2, 16), jnp.int32),
            pltpu.VMEM((SH, F), jnp.float32),
            pltpu.VMEM((SH, F), jnp.float32),
            pltpu.VMEM((SH, QF), jnp.float32),
            pltpu.VMEM((SH, QF), jnp.float32),
            pltpu.VMEM((SH, QF), jnp.float32),
            pltpu.VMEM((SH, QF), jnp.float32),
            pltpu.SemaphoreType.DMA((RING,)),
            pltpu.SemaphoreType.DMA((RING,)),
        ],
    )
    return f(h, rowi, coli, flg)[0]


def kernel(x, edge_index, W1, b1, W2, b2):
    h = _mlp(x, W1, b1, W2, b2)

    row = edge_index[0].reshape(NSC, E // NSC)
    col = edge_index[1].reshape(NSC, E // NSC)
    pad = jnp.full((NSC, EPT_PAD - E // NSC), N, dtype=jnp.int32)
    rowi = jnp.concatenate([row, pad], axis=1).reshape(NSC, NCH, C)
    coli = jnp.concatenate([col, pad], axis=1).reshape(NSC, NCH, C)

    flg = jnp.zeros((4, 16), jnp.int32)
    xk = _propagate(h, rowi, coli, flg)
    return _log_softmax(xk)
